# baseline (speedup 1.0000x reference)
import functools

import jax
import jax.numpy as jnp
from jax import lax
from jax.experimental import pallas as pl
from jax.experimental.pallas import tpu as pltpu
from jax.experimental.pallas import tpu_sc as plsc

VS = 32
B, N, D = 8, 4096, 512
NC, NS = 2, 16
NW = NC * NS
WPB = NW // B
DW = 64
PASSES = D // (WPB * DW)
CHUNK = 128
NCHUNK = N // CHUNK
LANES = 16
V = VS * VS


def _body(feat_hbm, xs_hbm, ys_hbm, out_hbm, xbuf, ybuf, idx_v, acc_a, acc_b,
          fbuf0, fbuf1, sem0, sem1):
    wid = lax.axis_index("s") * NC + lax.axis_index("c")
    b = wid // WPB
    j = wid % WPB

    pltpu.sync_copy(xs_hbm.at[b], xbuf)
    pltpu.sync_copy(ys_hbm.at[b], ybuf)

    def cidx(g, _):
        r = g // 8
        col = (g % 8) * LANES
        x = xbuf[r, pl.ds(col, LANES)]
        y = ybuf[r, pl.ds(col, LANES)]
        gx = jnp.clip((x * float(VS - 1)).astype(jnp.int32), 0, VS - 1)
        gy = jnp.clip((y * float(VS - 1)).astype(jnp.int32), 0, VS - 1)
        idx_v[pl.ds(g * LANES, LANES)] = gy * VS + gx
        return 0

    lax.fori_loop(0, N // LANES, cidx, 0)

    ninf16 = jnp.full((2 * LANES,), -jnp.inf, jnp.bfloat16)
    fbufs = (fbuf0, fbuf1)
    sems = (sem0, sem1)
    KG = DW // (2 * LANES)

    for p in range(PASSES):
        d0 = j * (PASSES * DW) + p * DW

        def initf(r, _):
            for k in range(KG):
                acc_a[pl.ds(r * DW + k * 2 * LANES, 2 * LANES)] = ninf16
                acc_b[pl.ds(r * DW + k * 2 * LANES, 2 * LANES)] = ninf16
            return 0

        lax.fori_loop(0, V, initf, 0)

        def feat_copy(c, buf, sem):
            return pltpu.make_async_copy(
                feat_hbm.at[b, pl.ds(c * CHUNK, CHUNK), pl.ds(d0, DW)],
                buf, sem)

        def process(c, fbuf):
            def pt(g, _):
                iv = idx_v[pl.ds(c * CHUNK + g * LANES, LANES)]
                for q in range(LANES):
                    i = iv[q]
                    n = g * LANES + q
                    acc = acc_a if q % 2 == 0 else acc_b
                    for k in range(KG):
                        flo = fbuf[n, pl.ds(k * 2 * LANES, LANES)]
                        fhi = fbuf[n, pl.ds(k * 2 * LANES + LANES, LANES)]
                        fm = plsc.pack(flo, fhi,
                                       format=plsc.PackFormat.INTERLEAVED)
                        sl = pl.ds(i * DW + k * 2 * LANES, 2 * LANES)
                        acc[sl] = jnp.maximum(acc[sl], fm)
                return 0

            lax.fori_loop(0, CHUNK // LANES, pt, 0)

        feat_copy(0, fbufs[0], sems[0]).start()
        feat_copy(1, fbufs[1], sems[1]).start()

        def chunk_pair(c2, _):
            c = c2 * 2
            feat_copy(0, fbufs[0], sems[0]).wait()
            process(c, fbufs[0])

            @pl.when(c2 + 1 < NCHUNK // 2)
            def _():
                feat_copy(c + 2, fbufs[0], sems[0]).start()

            feat_copy(0, fbufs[1], sems[1]).wait()
            process(c + 1, fbufs[1])

            @pl.when(c2 + 1 < NCHUNK // 2)
            def _():
                feat_copy(c + 3, fbufs[1], sems[1]).start()

            return 0

        lax.fori_loop(0, NCHUNK // 2, chunk_pair, 0)

        ROWS = CHUNK
        NBLK = V // ROWS

        def out_copy(blk, buf, sem):
            return pltpu.make_async_copy(
                buf, out_hbm.at[b, pl.ds(blk * ROWS, ROWS), pl.ds(d0, DW)],
                sem)

        def fill_block(blk, buf):
            def row(r, _):
                for k in range(KG):
                    sl = pl.ds((blk * ROWS + r) * DW + k * 2 * LANES,
                               2 * LANES)
                    m = jnp.maximum(acc_a[sl], acc_b[sl])
                    lo, hi = plsc.unpack(m, format=plsc.PackFormat.INTERLEAVED)
                    lo = jnp.where(lo == -jnp.inf, 0.0, lo)
                    hi = jnp.where(hi == -jnp.inf, 0.0, hi)
                    buf[r, pl.ds(k * 2 * LANES, LANES)] = lo
                    buf[r, pl.ds(k * 2 * LANES + LANES, LANES)] = hi
                return 0

            lax.fori_loop(0, ROWS, row, 0)

        for blk in range(NBLK):
            buf = fbufs[blk % 2]
            sem = sems[blk % 2]
            if blk >= 2:
                out_copy(0, buf, sem).wait()
            fill_block(blk, buf)
            out_copy(blk, buf, sem).start()
        out_copy(0, fbufs[0], sems[0]).wait()
        out_copy(0, fbufs[1], sems[1]).wait()


@jax.jit
def kernel(local_features, keypoint_coords):
    xs = keypoint_coords[:, :, 0].reshape(B, N // 128, 128)
    ys = keypoint_coords[:, :, 1].reshape(B, N // 128, 128)
    mesh = plsc.VectorSubcoreMesh(core_axis_name="c", subcore_axis_name="s",
                                  num_cores=NC, num_subcores=NS)
    out = pl.kernel(
        _body,
        out_type=jax.ShapeDtypeStruct((B, V, D), jnp.float32),
        mesh=mesh,
        compiler_params=pltpu.CompilerParams(use_tc_tiling_on_sc=False,
                                             needs_layout_passes=False),
        scratch_types=[
            pltpu.VMEM((N // 128, 128), jnp.float32),
            pltpu.VMEM((N // 128, 128), jnp.float32),
            pltpu.VMEM((N,), jnp.int32),
            pltpu.VMEM((V * DW,), jnp.bfloat16),
            pltpu.VMEM((V * DW,), jnp.bfloat16),
            pltpu.VMEM((CHUNK, DW), jnp.float32),
            pltpu.VMEM((CHUNK, DW), jnp.float32),
            pltpu.SemaphoreType.DMA,
            pltpu.SemaphoreType.DMA,
        ],
    )(local_features, xs, ys)
    return out.reshape(B, VS, VS, D)


# speedup vs baseline: 1.4355x; 1.4355x over previous
import functools

import jax
import jax.numpy as jnp
from jax import lax
from jax.experimental import pallas as pl
from jax.experimental.pallas import tpu as pltpu
from jax.experimental.pallas import tpu_sc as plsc

VS = 32
B, N, D = 8, 4096, 512
NC, NS = 2, 16
NW = NC * NS
WPB = NW // B
DW = 128
CHUNK = 128
NCHUNK = N // CHUNK
LANES = 16
V = VS * VS
RT = CHUNK // 8


def _body(feat_hbm, xs_hbm, ys_hbm, out_hbm, xbuf, ybuf, idx_v, acc_v,
          fbuf0, fbuf1, sem0, sem1):
    wid = lax.axis_index("s") * NC + lax.axis_index("c")
    b = wid // WPB
    q = wid % WPB
    d0 = q * DW

    pltpu.sync_copy(xs_hbm.at[b], xbuf)
    pltpu.sync_copy(ys_hbm.at[b], ybuf)

    def cidx(g, _):
        r = g // 8
        col = (g % 8) * LANES
        x = xbuf[r, pl.ds(col, LANES)]
        y = ybuf[r, pl.ds(col, LANES)]
        gx = jnp.clip((x * float(VS - 1)).astype(jnp.int32), 0, VS - 1)
        gy = jnp.clip((y * float(VS - 1)).astype(jnp.int32), 0, VS - 1)
        idx_v[pl.ds(g * LANES, LANES)] = gy * VS + gx
        return 0

    lax.fori_loop(0, N // LANES, cidx, 0)

    ninf16 = jnp.full((2 * LANES,), -jnp.inf, jnp.bfloat16)
    KG = DW // (2 * LANES)

    def initf(r, _):
        for k in range(KG):
            acc_v[pl.ds(r * DW + k * 2 * LANES, 2 * LANES)] = ninf16
        return 0

    lax.fori_loop(0, V, initf, 0)

    fbufs = (fbuf0, fbuf1)
    sems = (sem0, sem1)

    def feat_copy(c, buf, sem):
        return pltpu.make_async_copy(
            feat_hbm.at[b, pl.ds(c * RT, RT), q], buf, sem)

    def process(c, fbuf):
        def pt(g, _):
            iv = idx_v[pl.ds(c * CHUNK + g * LANES, LANES)]
            for qq in range(LANES):
                i = iv[qq]
                rt = g * 2 + qq // 8
                sr = qq % 8
                for k in range(KG):
                    flo = fbuf[rt, sr, pl.ds(k * 2 * LANES, LANES)]
                    fhi = fbuf[rt, sr, pl.ds(k * 2 * LANES + LANES, LANES)]
                    fm = plsc.pack(flo, fhi,
                                   format=plsc.PackFormat.INTERLEAVED)
                    sl = pl.ds(i * DW + k * 2 * LANES, 2 * LANES)
                    acc_v[sl] = jnp.maximum(acc_v[sl], fm)
            return 0

        lax.fori_loop(0, CHUNK // LANES, pt, 0)

    feat_copy(0, fbufs[0], sems[0]).start()
    feat_copy(1, fbufs[1], sems[1]).start()

    def chunk_pair(c2, _):
        c = c2 * 2
        feat_copy(0, fbufs[0], sems[0]).wait()
        process(c, fbufs[0])

        @pl.when(c2 + 1 < NCHUNK // 2)
        def _():
            feat_copy(c + 2, fbufs[0], sems[0]).start()

        feat_copy(0, fbufs[1], sems[1]).wait()
        process(c + 1, fbufs[1])

        @pl.when(c2 + 1 < NCHUNK // 2)
        def _():
            feat_copy(c + 3, fbufs[1], sems[1]).start()

        return 0

    lax.fori_loop(0, NCHUNK // 2, chunk_pair, 0)

    ROWS = 128
    NBLK = V // ROWS

    def out_copy(blk, buf, sem):
        return pltpu.make_async_copy(
            buf, out_hbm.at[b, pl.ds(blk * RT, RT), :, pl.ds(d0, DW)], sem)

    def fill_block(blk, buf):
        def row(r, _):
            rt = r // 8
            sr = r % 8
            for k in range(KG):
                v = acc_v[pl.ds((blk * ROWS + r) * DW + k * 2 * LANES,
                                2 * LANES)]
                lo, hi = plsc.unpack(v, format=plsc.PackFormat.INTERLEAVED)
                lo = jnp.where(lo == -jnp.inf, 0.0, lo)
                hi = jnp.where(hi == -jnp.inf, 0.0, hi)
                buf[rt, sr, pl.ds(k * 2 * LANES, LANES)] = lo
                buf[rt, sr, pl.ds(k * 2 * LANES + LANES, LANES)] = hi
            return 0

        lax.fori_loop(0, ROWS, row, 0)

    for blk in range(NBLK):
        buf = fbufs[blk % 2]
        sem = sems[blk % 2]
        if blk >= 2:
            out_copy(0, buf, sem).wait()
        fill_block(blk, buf)
        out_copy(blk, buf, sem).start()
    out_copy(0, fbufs[0], sems[0]).wait()
    out_copy(0, fbufs[1], sems[1]).wait()


@jax.jit
def kernel(local_features, keypoint_coords):
    ft = local_features.reshape(B, N // 8, 8, D // 128, 128)
    ft = ft.transpose(0, 1, 3, 2, 4)
    xs = keypoint_coords[:, :, 0].reshape(B, N // 128, 128)
    ys = keypoint_coords[:, :, 1].reshape(B, N // 128, 128)
    mesh = plsc.VectorSubcoreMesh(core_axis_name="c", subcore_axis_name="s",
                                  num_cores=NC, num_subcores=NS)
    out = pl.kernel(
        _body,
        out_type=jax.ShapeDtypeStruct((B, V // 8, 8, D), jnp.float32),
        mesh=mesh,
        compiler_params=pltpu.CompilerParams(use_tc_tiling_on_sc=False,
                                             needs_layout_passes=False),
        scratch_types=[
            pltpu.VMEM((N // 128, 128), jnp.float32),
            pltpu.VMEM((N // 128, 128), jnp.float32),
            pltpu.VMEM((N,), jnp.int32),
            pltpu.VMEM((V * DW,), jnp.bfloat16),
            pltpu.VMEM((RT, 8, 128), jnp.float32),
            pltpu.VMEM((RT, 8, 128), jnp.float32),
            pltpu.SemaphoreType.DMA,
            pltpu.SemaphoreType.DMA,
        ],
    )(ft, xs, ys)
    return out.reshape(B, VS, VS, D)


# speedup vs baseline: 1.4449x; 1.0065x over previous
import functools

import jax
import jax.numpy as jnp
from jax import lax
from jax.experimental import pallas as pl
from jax.experimental.pallas import tpu as pltpu
from jax.experimental.pallas import tpu_sc as plsc

VS = 32
B, N, D = 8, 4096, 512
NC, NS = 2, 16
NW = NC * NS
WPB = NW // B
DW = 128
CHUNK = 128
NCHUNK = N // CHUNK
LANES = 16
V = VS * VS
RT = CHUNK // 8


def _body(feat_hbm, xs_hbm, ys_hbm, out_hbm, xbuf, ybuf, idx_v, acc_v,
          fbuf0, fbuf1, sem0, sem1):
    wid = lax.axis_index("s") * NC + lax.axis_index("c")
    b = wid // WPB
    q = wid % WPB
    d0 = q * DW

    pltpu.sync_copy(xs_hbm.at[b], xbuf)
    pltpu.sync_copy(ys_hbm.at[b], ybuf)

    def cidx(g, _):
        r = g // 8
        col = (g % 8) * LANES
        x = xbuf[r, pl.ds(col, LANES)]
        y = ybuf[r, pl.ds(col, LANES)]
        gx = jnp.clip((x * float(VS - 1)).astype(jnp.int32), 0, VS - 1)
        gy = jnp.clip((y * float(VS - 1)).astype(jnp.int32), 0, VS - 1)
        idx_v[pl.ds(g * LANES, LANES)] = (gy * VS + gx) * DW
        return 0

    lax.fori_loop(0, N // LANES, cidx, 0)

    ninf16 = jnp.full((2 * LANES,), -jnp.inf, jnp.bfloat16)
    KG = DW // (2 * LANES)
    IUNROLL = 4

    def initf(r4, _):
        for r in range(IUNROLL):
            for k in range(KG):
                acc_v[pl.ds((r4 * IUNROLL + r) * DW + k * 2 * LANES,
                            2 * LANES)] = ninf16
        return 0

    lax.fori_loop(0, V // IUNROLL, initf, 0)

    fbufs = (fbuf0, fbuf1)
    sems = (sem0, sem1)

    def feat_copy(c, buf, sem):
        return pltpu.make_async_copy(
            feat_hbm.at[b, pl.ds(c * RT, RT), q], buf, sem)

    def process(c, fbuf):
        def pt(g, _):
            iv = idx_v[pl.ds(c * CHUNK + g * LANES, LANES)]
            for qq in range(LANES):
                i = iv[qq]
                rt = g * 2 + qq // 8
                sr = qq % 8
                for k in range(KG):
                    flo = fbuf[rt, sr, pl.ds(k * 2 * LANES, LANES)]
                    fhi = fbuf[rt, sr, pl.ds(k * 2 * LANES + LANES, LANES)]
                    fm = plsc.pack(flo, fhi,
                                   format=plsc.PackFormat.INTERLEAVED)
                    sl = pl.ds(i + k * 2 * LANES, 2 * LANES)
                    acc_v[sl] = jnp.maximum(acc_v[sl], fm)
            return 0

        lax.fori_loop(0, CHUNK // LANES, pt, 0)

    feat_copy(0, fbufs[0], sems[0]).start()
    feat_copy(1, fbufs[1], sems[1]).start()

    def chunk_pair(c2, _):
        c = c2 * 2
        feat_copy(0, fbufs[0], sems[0]).wait()
        process(c, fbufs[0])

        @pl.when(c2 + 1 < NCHUNK // 2)
        def _():
            feat_copy(c + 2, fbufs[0], sems[0]).start()

        feat_copy(0, fbufs[1], sems[1]).wait()
        process(c + 1, fbufs[1])

        @pl.when(c2 + 1 < NCHUNK // 2)
        def _():
            feat_copy(c + 3, fbufs[1], sems[1]).start()

        return 0

    lax.fori_loop(0, NCHUNK // 2, chunk_pair, 0)

    ROWS = 128
    NBLK = V // ROWS

    def out_copy(blk, buf, sem):
        return pltpu.make_async_copy(
            buf, out_hbm.at[b, pl.ds(blk * RT, RT), :, pl.ds(d0, DW)], sem)

    def fill_block(blk, buf):
        def row(r2, _):
            for h in range(2):
                r = r2 * 2 + h
                rt = r // 8
                sr = r % 8
                for k in range(KG):
                    v = acc_v[pl.ds((blk * ROWS + r) * DW + k * 2 * LANES,
                                    2 * LANES)]
                    lo, hi = plsc.unpack(v,
                                         format=plsc.PackFormat.INTERLEAVED)
                    lo = jnp.where(lo == -jnp.inf, 0.0, lo)
                    hi = jnp.where(hi == -jnp.inf, 0.0, hi)
                    buf[rt, sr, pl.ds(k * 2 * LANES, LANES)] = lo
                    buf[rt, sr, pl.ds(k * 2 * LANES + LANES, LANES)] = hi
            return 0

        lax.fori_loop(0, ROWS // 2, row, 0)

    for blk in range(NBLK):
        buf = fbufs[blk % 2]
        sem = sems[blk % 2]
        if blk >= 2:
            out_copy(0, buf, sem).wait()
        fill_block(blk, buf)
        out_copy(blk, buf, sem).start()
    out_copy(0, fbufs[0], sems[0]).wait()
    out_copy(0, fbufs[1], sems[1]).wait()


@jax.jit
def kernel(local_features, keypoint_coords):
    ft = local_features.reshape(B, N // 8, 8, D // 128, 128)
    ft = ft.transpose(0, 1, 3, 2, 4)
    xs = keypoint_coords[:, :, 0].reshape(B, N // 128, 128)
    ys = keypoint_coords[:, :, 1].reshape(B, N // 128, 128)
    mesh = plsc.VectorSubcoreMesh(core_axis_name="c", subcore_axis_name="s",
                                  num_cores=NC, num_subcores=NS)
    out = pl.kernel(
        _body,
        out_type=jax.ShapeDtypeStruct((B, V // 8, 8, D), jnp.float32),
        mesh=mesh,
        compiler_params=pltpu.CompilerParams(use_tc_tiling_on_sc=False,
                                             needs_layout_passes=False),
        scratch_types=[
            pltpu.VMEM((N // 128, 128), jnp.float32),
            pltpu.VMEM((N // 128, 128), jnp.float32),
            pltpu.VMEM((N,), jnp.int32),
            pltpu.VMEM((V * DW,), jnp.bfloat16),
            pltpu.VMEM((RT, 8, 128), jnp.float32),
            pltpu.VMEM((RT, 8, 128), jnp.float32),
            pltpu.SemaphoreType.DMA,
            pltpu.SemaphoreType.DMA,
        ],
    )(ft, xs, ys)
    return out.reshape(B, VS, VS, D)


# speedup vs baseline: 1.4535x; 1.0059x over previous
import functools

import jax
import jax.numpy as jnp
from jax import lax
from jax.experimental import pallas as pl
from jax.experimental.pallas import tpu as pltpu
from jax.experimental.pallas import tpu_sc as plsc

VS = 32
B, N, D = 8, 4096, 512
NC, NS = 2, 16
NW = NC * NS
WPB = NW // B
DW = 128
CHUNK = 128
NCHUNK = N // CHUNK
LANES = 16
V = VS * VS
RT = CHUNK // 8


def _body(feat_hbm, xs_hbm, ys_hbm, out_hbm, xbuf, ybuf, idx_v, acc_v,
          fbuf0, fbuf1, sem0, sem1):
    wid = lax.axis_index("s") * NC + lax.axis_index("c")
    b = wid // WPB
    q = wid % WPB
    d0 = q * DW

    fbufs = (fbuf0, fbuf1)
    sems = (sem0, sem1)

    def feat_copy(c, buf, sem):
        return pltpu.make_async_copy(
            feat_hbm.at[b, pl.ds(c * RT, RT), q], buf, sem)

    feat_copy(0, fbufs[0], sems[0]).start()
    feat_copy(1, fbufs[1], sems[1]).start()

    pltpu.sync_copy(xs_hbm.at[b], xbuf)
    pltpu.sync_copy(ys_hbm.at[b], ybuf)

    def cidx(g, _):
        r = g // 8
        col = (g % 8) * LANES
        x = xbuf[r, pl.ds(col, LANES)]
        y = ybuf[r, pl.ds(col, LANES)]
        gx = jnp.clip((x * float(VS - 1)).astype(jnp.int32), 0, VS - 1)
        gy = jnp.clip((y * float(VS - 1)).astype(jnp.int32), 0, VS - 1)
        idx_v[pl.ds(g * LANES, LANES)] = (gy * VS + gx) * DW
        return 0

    lax.fori_loop(0, N // LANES, cidx, 0)

    ninf16 = jnp.full((2 * LANES,), -jnp.inf, jnp.bfloat16)
    KG = DW // (2 * LANES)
    IUNROLL = 4

    def initf(r4, _):
        for r in range(IUNROLL):
            for k in range(KG):
                acc_v[pl.ds((r4 * IUNROLL + r) * DW + k * 2 * LANES,
                            2 * LANES)] = ninf16
        return 0

    lax.fori_loop(0, V // IUNROLL, initf, 0)

    def process(c, fbuf):
        def pt(g, _):
            iv = idx_v[pl.ds(c * CHUNK + g * LANES, LANES)]
            for qq in range(LANES):
                i = iv[qq]
                rt = g * 2 + qq // 8
                sr = qq % 8
                for k in range(KG):
                    flo = fbuf[rt, sr, pl.ds(k * 2 * LANES, LANES)]
                    fhi = fbuf[rt, sr, pl.ds(k * 2 * LANES + LANES, LANES)]
                    fm = plsc.pack(flo, fhi,
                                   format=plsc.PackFormat.INTERLEAVED)
                    sl = pl.ds(i + k * 2 * LANES, 2 * LANES)
                    acc_v[sl] = jnp.maximum(acc_v[sl], fm)
            return 0

        lax.fori_loop(0, CHUNK // LANES, pt, 0)

    def chunk_pair(c2, _):
        c = c2 * 2
        feat_copy(0, fbufs[0], sems[0]).wait()
        process(c, fbufs[0])

        @pl.when(c2 + 1 < NCHUNK // 2)
        def _():
            feat_copy(c + 2, fbufs[0], sems[0]).start()

        feat_copy(0, fbufs[1], sems[1]).wait()
        process(c + 1, fbufs[1])

        @pl.when(c2 + 1 < NCHUNK // 2)
        def _():
            feat_copy(c + 3, fbufs[1], sems[1]).start()

        return 0

    lax.fori_loop(0, NCHUNK // 2, chunk_pair, 0)

    ROWS = 128
    NBLK = V // ROWS

    def out_copy(blk, buf, sem):
        return pltpu.make_async_copy(
            buf, out_hbm.at[b, pl.ds(blk * RT, RT), :, pl.ds(d0, DW)], sem)

    def fill_block(blk, buf):
        def row(r2, _):
            for h in range(2):
                r = r2 * 2 + h
                rt = r // 8
                sr = r % 8
                for k in range(KG):
                    v = acc_v[pl.ds((blk * ROWS + r) * DW + k * 2 * LANES,
                                    2 * LANES)]
                    lo, hi = plsc.unpack(v,
                                         format=plsc.PackFormat.INTERLEAVED)
                    lo = jnp.where(lo == -jnp.inf, 0.0, lo)
                    hi = jnp.where(hi == -jnp.inf, 0.0, hi)
                    buf[rt, sr, pl.ds(k * 2 * LANES, LANES)] = lo
                    buf[rt, sr, pl.ds(k * 2 * LANES + LANES, LANES)] = hi
            return 0

        lax.fori_loop(0, ROWS // 2, row, 0)

    for blk in range(NBLK):
        buf = fbufs[blk % 2]
        sem = sems[blk % 2]
        if blk >= 2:
            out_copy(0, buf, sem).wait()
        fill_block(blk, buf)
        out_copy(blk, buf, sem).start()
    out_copy(0, fbufs[0], sems[0]).wait()
    out_copy(0, fbufs[1], sems[1]).wait()


@jax.jit
def kernel(local_features, keypoint_coords):
    ft = local_features.reshape(B, N // 8, 8, D // 128, 128)
    ft = ft.transpose(0, 1, 3, 2, 4)
    xs = keypoint_coords[:, :, 0].reshape(B, N // 128, 128)
    ys = keypoint_coords[:, :, 1].reshape(B, N // 128, 128)
    mesh = plsc.VectorSubcoreMesh(core_axis_name="c", subcore_axis_name="s",
                                  num_cores=NC, num_subcores=NS)
    out = pl.kernel(
        _body,
        out_type=jax.ShapeDtypeStruct((B, V // 8, 8, D), jnp.float32),
        mesh=mesh,
        compiler_params=pltpu.CompilerParams(use_tc_tiling_on_sc=False,
                                             needs_layout_passes=False),
        scratch_types=[
            pltpu.VMEM((N // 128, 128), jnp.float32),
            pltpu.VMEM((N // 128, 128), jnp.float32),
            pltpu.VMEM((N,), jnp.int32),
            pltpu.VMEM((V * DW,), jnp.bfloat16),
            pltpu.VMEM((RT, 8, 128), jnp.float32),
            pltpu.VMEM((RT, 8, 128), jnp.float32),
            pltpu.SemaphoreType.DMA,
            pltpu.SemaphoreType.DMA,
        ],
    )(ft, xs, ys)
    return out.reshape(B, VS, VS, D)
